# baseline (speedup 1.0000x reference)
import functools

import jax
import jax.numpy as jnp
from jax import lax
from jax.experimental import pallas as pl
from jax.experimental.pallas import tpu as pltpu
from jax.experimental.pallas import tpu_sc as plsc

_NN = 10000
_NE = 320000
_D = 128
_NP = 10240
_NC = 2
_NS = 16
_NW = _NC * _NS
_EPW = 10240
_EP = _NW * _EPW
_K = 128
_CPW = _EPW // _K
_KH = 64
_G = 4
_PC = 16
_QP = 2 * _PC
_NPH = _CPW // _PC
_RPT = _NP // _NS

_mesh = plsc.VectorSubcoreMesh(core_axis_name="c", subcore_axis_name="s")


@functools.partial(
    pl.kernel,
    out_type=jax.ShapeDtypeStruct((_NC, _NP), jnp.float32),
    mesh=_mesh,
    scratch_types=[
        pltpu.VMEM((_NP,), jnp.float32),
        pltpu.VMEM((_CPW, _K), jnp.int32),
        pltpu.VMEM((_NS, _NP // _NS), jnp.float32),
        pltpu.VMEM((_NP // _NS,), jnp.float32),
        pltpu.VMEM_SHARED((_NS, _NP), jnp.float32),
    ],
    compiler_params=pltpu.CompilerParams(needs_layout_passes=False),
)
def _sc_degree(dst_hbm, deg_out, deg_v, dst_v, red_v, sum_v, shared):
    cid = lax.axis_index("c")
    sid = lax.axis_index("s")
    wid = sid * _NC + cid
    cpt = _NP // _NS

    zeros16 = jnp.zeros((16,), jnp.float32)

    def zbody(i, c):
        deg_v[pl.ds(i * 16, 16)] = zeros16
        return c

    lax.fori_loop(0, _NP // 16, zbody, 0)

    pltpu.sync_copy(dst_hbm.at[pl.ds(wid * _CPW, _CPW)], dst_v)

    ones16 = jnp.ones((16,), jnp.float32)

    def chunk(i, c):
        for j in range(_K // 16):
            idx = dst_v[i, pl.ds(j * 16, 16)]
            plsc.addupdate_scatter(deg_v, [idx], ones16)
        return c

    lax.fori_loop(0, _CPW, chunk, 0)

    pltpu.sync_copy(deg_v, shared.at[sid])
    plsc.subcore_barrier()
    for p in range(_NS):
        pltpu.sync_copy(shared.at[p, pl.ds(sid * cpt, cpt)], red_v.at[p])
    for j in range(cpt // 16):
        acc = zeros16
        for p in range(_NS):
            acc = acc + red_v[p, pl.ds(j * 16, 16)]
        sum_v[pl.ds(j * 16, 16)] = acc
    pltpu.sync_copy(sum_v, deg_out.at[cid, pl.ds(sid * cpt, cpt)])


@functools.partial(
    pl.kernel,
    out_type=jax.ShapeDtypeStruct((_NC, _NP, _D), jnp.float32),
    mesh=_mesh,
    scratch_types=[
        pltpu.VMEM((2, _PC, _K), jnp.int32),
        pltpu.VMEM((2, _PC, _K), jnp.int32),
        pltpu.VMEM((_G, _KH, _D), jnp.float32),
        pltpu.VMEM_SHARED((_NP, _D), jnp.float32),
        pltpu.SemaphoreType.DMA((_G,)),
        pltpu.SemaphoreType.DMA((_G,)),
        pltpu.SemaphoreType.DMA,
    ],
    compiler_params=pltpu.CompilerParams(needs_layout_passes=False),
)
def _sc_scatter(g_hbm, src_hbm, dst_hbm, zeros_hbm, out_hbm,
                sidx, didx, rows, accum, gsem, ssem, isem):
    cid = lax.axis_index("c")
    sid = lax.axis_index("s")
    wid = sid * _NC + cid

    pltpu.sync_copy(zeros_hbm.at[pl.ds(sid * _RPT, _RPT)],
                    accum.at[pl.ds(sid * _RPT, _RPT)])

    rbase = wid * _CPW
    pltpu.sync_copy(src_hbm.at[pl.ds(rbase, _PC)], sidx.at[0])
    pltpu.sync_copy(dst_hbm.at[pl.ds(rbase, _PC)], didx.at[0])
    plsc.subcore_barrier()

    def phase(p, c):
        buf = lax.rem(p, 2)
        nxt = lax.rem(p + 1, 2)

        @pl.when(p > 0)
        def _():
            pltpu.make_async_copy(
                src_hbm.at[pl.ds(rbase + p * _PC, _PC)], sidx.at[buf],
                isem).wait()
            pltpu.make_async_copy(
                dst_hbm.at[pl.ds(rbase + p * _PC, _PC)], didx.at[buf],
                isem).wait()

        @pl.when(p < _NPH - 1)
        def _():
            row0 = rbase + (p + 1) * _PC
            pltpu.async_copy(src_hbm.at[pl.ds(row0, _PC)], sidx.at[nxt], isem)
            pltpu.async_copy(dst_hbm.at[pl.ds(row0, _PC)], didx.at[nxt], isem)

        def src_at(q):
            return src_idx_slice(sidx, buf, q)

        def dst_at(q):
            return dst_idx_slice(didx, buf, q)

        for j in range(_G):
            pltpu.async_copy(g_hbm.at[src_at(j)], rows.at[j], gsem.at[j])

        def chunk(q, c2):
            slot = lax.rem(q, _G)
            pltpu.make_async_copy(
                g_hbm.at[src_at(q)], rows.at[slot], gsem.at[slot]).wait()
            pltpu.async_copy(
                rows.at[slot], accum.at[dst_at(q)], ssem.at[slot], add=True)

            pq = q - 1
            @pl.when(jnp.logical_and(q >= 1, pq + _G < _QP))
            def _():
                ps = lax.rem(pq, _G)
                pltpu.make_async_copy(
                    rows.at[ps], accum.at[dst_at(pq)], ssem.at[ps]).wait()
                pltpu.async_copy(
                    g_hbm.at[src_at(pq + _G)], rows.at[ps], gsem.at[ps])
            return c2

        lax.fori_loop(0, _QP, chunk, 0)
        for j in range(_QP - _G, _QP):
            pltpu.make_async_copy(
                rows.at[j % _G], accum.at[dst_at(j)], ssem.at[j % _G]).wait()
        return c

    lax.fori_loop(0, _NPH, phase, 0)
    plsc.subcore_barrier()
    pltpu.sync_copy(accum.at[pl.ds(sid * _RPT, _RPT)],
                    out_hbm.at[cid, pl.ds(sid * _RPT, _RPT)])


def src_idx_slice(sidx, buf, q):
    return sidx.at[buf, lax.div(q, 2), pl.ds(lax.rem(q, 2) * _KH, _KH)]


def dst_idx_slice(didx, buf, q):
    return didx.at[buf, lax.div(q, 2), pl.ds(lax.rem(q, 2) * _KH, _KH)]


_R = 2000


def _dot(a, b):
    return lax.dot_general(a, b, (((1,), (0,)), ((), ())),
                           precision=lax.Precision.HIGHEST,
                           preferred_element_type=jnp.float32)


def _tc_scale_body(x_ref, w_ref, deg_ref, g_ref, dinv_ref):
    deg = jnp.sum(deg_ref[...], axis=0) + 1.0
    dinv = lax.rsqrt(deg)
    g_ref[...] = _dot(x_ref[...], w_ref[...]) * dinv
    dinv_ref[...] = dinv


def _tc_scale(x, W1, degp):
    return pl.pallas_call(
        _tc_scale_body,
        grid=(_NN // _R,),
        in_specs=[
            pl.BlockSpec((_R, _D), lambda i: (i, 0)),
            pl.BlockSpec((_D, _D), lambda i: (0, 0)),
            pl.BlockSpec((_NC, _R, 1), lambda i: (0, i, 0)),
        ],
        out_specs=[
            pl.BlockSpec((_R, _D), lambda i: (i, 0)),
            pl.BlockSpec((_R, 1), lambda i: (i, 0)),
        ],
        out_shape=[
            jax.ShapeDtypeStruct((_NN, _D), jnp.float32),
            jax.ShapeDtypeStruct((_NN, 1), jnp.float32),
        ],
    )(x, W1, degp)


def _tc2_body(s_ref, g_ref, dinv_ref, w_ref, g2_ref):
    dinv = dinv_ref[...]
    p = (s_ref[0] + s_ref[1] + g_ref[...]) * dinv
    g2_ref[...] = _dot(p, w_ref[...]) * dinv


def _tc_layer2(s, g, dinv, W2):
    return pl.pallas_call(
        _tc2_body,
        grid=(_NN // _R,),
        in_specs=[
            pl.BlockSpec((_NC, _R, _D), lambda i: (0, i, 0)),
            pl.BlockSpec((_R, _D), lambda i: (i, 0)),
            pl.BlockSpec((_R, 1), lambda i: (i, 0)),
            pl.BlockSpec((_D, _D), lambda i: (0, 0)),
        ],
        out_specs=pl.BlockSpec((_R, _D), lambda i: (i, 0)),
        out_shape=jax.ShapeDtypeStruct((_NN, _D), jnp.float32),
    )(s, g, dinv, W2)


def _tc3_body(s_ref, g_ref, dinv_ref, o_ref):
    o_ref[...] = (s_ref[0] + s_ref[1] + g_ref[...]) * dinv_ref[...]


def _tc_layer3(s, g, dinv):
    return pl.pallas_call(
        _tc3_body,
        grid=(_NN // _R,),
        in_specs=[
            pl.BlockSpec((_NC, _R, _D), lambda i: (0, i, 0)),
            pl.BlockSpec((_R, _D), lambda i: (i, 0)),
            pl.BlockSpec((_R, 1), lambda i: (i, 0)),
        ],
        out_specs=pl.BlockSpec((_R, _D), lambda i: (i, 0)),
        out_shape=jax.ShapeDtypeStruct((_NN, _D), jnp.float32),
    )(s, g, dinv)


def kernel(x, edge_index, W1, W2):
    src = edge_index[0]
    dst = edge_index[1]
    pidx = jnp.arange(_EP - _NE, dtype=jnp.int32)
    src_p = jnp.concatenate([src, pidx % _NN]).reshape(_EP // _K, _K)
    dst_p = jnp.concatenate([dst, _NN + pidx % (_NP - _NN)]).reshape(
        _EP // _K, _K)
    zeros = jnp.zeros((_NP, _D), jnp.float32)

    degp = _sc_degree(dst_p).reshape(_NC, _NP, 1)
    g1, dinv = _tc_scale(x, W1, degp)

    s1 = _sc_scatter(g1, src_p, dst_p, zeros)
    g2 = _tc_layer2(s1, g1, dinv, W2)

    s2 = _sc_scatter(g2, src_p, dst_p, zeros)
    return _tc_layer3(s2, g2, dinv)


# speedup vs baseline: 1.0374x; 1.0374x over previous
import functools

import jax
import jax.numpy as jnp
from jax import lax
from jax.experimental import pallas as pl
from jax.experimental.pallas import tpu as pltpu
from jax.experimental.pallas import tpu_sc as plsc

_NN = 10000
_NE = 320000
_D = 128
_NP = 10240
_NC = 2
_NS = 16
_NW = _NC * _NS
_EPW = 10240
_EP = _NW * _EPW
_K = 128
_CPW = _EPW // _K
_KH = 64
_G = 4
_PC = 16
_QP = 2 * _PC
_NPH = _CPW // _PC
_RPT = _NP // _NS

_mesh = plsc.VectorSubcoreMesh(core_axis_name="c", subcore_axis_name="s")


@functools.partial(
    pl.kernel,
    out_type=jax.ShapeDtypeStruct((_NC, _NP), jnp.float32),
    mesh=_mesh,
    scratch_types=[
        pltpu.VMEM((_NP,), jnp.float32),
        pltpu.VMEM((_CPW, _K), jnp.int32),
        pltpu.VMEM((_NS, _NP // _NS), jnp.float32),
        pltpu.VMEM((_NP // _NS,), jnp.float32),
        pltpu.VMEM_SHARED((_NS, _NP), jnp.float32),
    ],
    compiler_params=pltpu.CompilerParams(needs_layout_passes=False),
)
def _sc_degree(dst_hbm, deg_out, deg_v, dst_v, red_v, sum_v, shared):
    cid = lax.axis_index("c")
    sid = lax.axis_index("s")
    wid = sid * _NC + cid
    cpt = _NP // _NS

    zeros16 = jnp.zeros((16,), jnp.float32)

    def zbody(i, c):
        deg_v[pl.ds(i * 16, 16)] = zeros16
        return c

    lax.fori_loop(0, _NP // 16, zbody, 0)

    pltpu.sync_copy(dst_hbm.at[pl.ds(wid * _CPW, _CPW)], dst_v)

    ones16 = jnp.ones((16,), jnp.float32)

    def chunk(i, c):
        for j in range(_K // 16):
            idx = dst_v[i, pl.ds(j * 16, 16)]
            plsc.addupdate_scatter(deg_v, [idx], ones16)
        return c

    lax.fori_loop(0, _CPW, chunk, 0)

    pltpu.sync_copy(deg_v, shared.at[sid])
    plsc.subcore_barrier()
    for p in range(_NS):
        pltpu.sync_copy(shared.at[p, pl.ds(sid * cpt, cpt)], red_v.at[p])
    for j in range(cpt // 16):
        acc = zeros16
        for p in range(_NS):
            acc = acc + red_v[p, pl.ds(j * 16, 16)]
        sum_v[pl.ds(j * 16, 16)] = acc
    pltpu.sync_copy(sum_v, deg_out.at[cid, pl.ds(sid * cpt, cpt)])


@functools.partial(
    pl.kernel,
    out_type=jax.ShapeDtypeStruct((_NC, _NP, _D), jnp.float32),
    mesh=_mesh,
    scratch_types=[
        pltpu.VMEM((2, _PC, _K), jnp.int32),
        pltpu.VMEM((2, _PC, _K), jnp.int32),
        pltpu.VMEM((_G, _KH, _D), jnp.float32),
        pltpu.VMEM_SHARED((_NP, _D), jnp.float32),
        pltpu.SemaphoreType.DMA((_G,)),
        pltpu.SemaphoreType.DMA((_G,)),
        pltpu.SemaphoreType.DMA,
    ],
    compiler_params=pltpu.CompilerParams(needs_layout_passes=False),
)
def _sc_scatter(g_hbm, src_hbm, dst_hbm, out_hbm,
                sidx, didx, rows, accum, gsem, ssem, isem):
    cid = lax.axis_index("c")
    sid = lax.axis_index("s")
    wid = sid * _NC + cid

    zeros16 = jnp.zeros((16,), jnp.float32)

    def zbody(r, c):
        for j in range(_D // 16):
            rows[0, r, pl.ds(j * 16, 16)] = zeros16
        return c

    lax.fori_loop(0, _KH, zbody, 0)
    for t in range(_RPT // _KH):
        pltpu.async_copy(rows.at[0],
                         accum.at[pl.ds(sid * _RPT + t * _KH, _KH)], isem)

    rbase = wid * _CPW
    pltpu.sync_copy(src_hbm.at[pl.ds(rbase, _PC)], sidx.at[0])
    pltpu.sync_copy(dst_hbm.at[pl.ds(rbase, _PC)], didx.at[0])
    for t in range(_RPT // _KH):
        pltpu.make_async_copy(
            rows.at[0], accum.at[pl.ds(sid * _RPT + t * _KH, _KH)],
            isem).wait()
    plsc.subcore_barrier()

    def phase(p, c):
        buf = lax.rem(p, 2)
        nxt = lax.rem(p + 1, 2)

        @pl.when(p > 0)
        def _():
            pltpu.make_async_copy(
                src_hbm.at[pl.ds(rbase + p * _PC, _PC)], sidx.at[buf],
                isem).wait()
            pltpu.make_async_copy(
                dst_hbm.at[pl.ds(rbase + p * _PC, _PC)], didx.at[buf],
                isem).wait()

        @pl.when(p < _NPH - 1)
        def _():
            row0 = rbase + (p + 1) * _PC
            pltpu.async_copy(src_hbm.at[pl.ds(row0, _PC)], sidx.at[nxt], isem)
            pltpu.async_copy(dst_hbm.at[pl.ds(row0, _PC)], didx.at[nxt], isem)

        def src_at(q):
            return src_idx_slice(sidx, buf, q)

        def dst_at(q):
            return dst_idx_slice(didx, buf, q)

        for j in range(_G):
            pltpu.async_copy(g_hbm.at[src_at(j)], rows.at[j], gsem.at[j])

        def chunk(q, c2):
            slot = lax.rem(q, _G)
            pltpu.make_async_copy(
                g_hbm.at[src_at(q)], rows.at[slot], gsem.at[slot]).wait()
            pltpu.async_copy(
                rows.at[slot], accum.at[dst_at(q)], ssem.at[slot], add=True)

            pq = q - 1
            @pl.when(jnp.logical_and(q >= 1, pq + _G < _QP))
            def _():
                ps = lax.rem(pq, _G)
                pltpu.make_async_copy(
                    rows.at[ps], accum.at[dst_at(pq)], ssem.at[ps]).wait()
                pltpu.async_copy(
                    g_hbm.at[src_at(pq + _G)], rows.at[ps], gsem.at[ps])
            return c2

        lax.fori_loop(0, _QP, chunk, 0)
        for j in range(_QP - _G, _QP):
            pltpu.make_async_copy(
                rows.at[j % _G], accum.at[dst_at(j)], ssem.at[j % _G]).wait()
        return c

    lax.fori_loop(0, _NPH, phase, 0)
    plsc.subcore_barrier()
    pltpu.sync_copy(accum.at[pl.ds(sid * _RPT, _RPT)],
                    out_hbm.at[cid, pl.ds(sid * _RPT, _RPT)])


def src_idx_slice(sidx, buf, q):
    return sidx.at[buf, lax.div(q, 2), pl.ds(lax.rem(q, 2) * _KH, _KH)]


def dst_idx_slice(didx, buf, q):
    return didx.at[buf, lax.div(q, 2), pl.ds(lax.rem(q, 2) * _KH, _KH)]


_R = 2000


def _dot(a, b):
    return lax.dot_general(a, b, (((1,), (0,)), ((), ())),
                           precision=lax.Precision.HIGHEST,
                           preferred_element_type=jnp.float32)


def _tc_scale_body(x_ref, w_ref, deg_ref, g_ref, dinv_ref):
    deg = jnp.sum(deg_ref[...], axis=0) + 1.0
    dinv = lax.rsqrt(deg)
    g_ref[...] = _dot(x_ref[...], w_ref[...]) * dinv
    dinv_ref[...] = dinv


def _tc_scale(x, W1, degp):
    return pl.pallas_call(
        _tc_scale_body,
        grid=(_NN // _R,),
        in_specs=[
            pl.BlockSpec((_R, _D), lambda i: (i, 0)),
            pl.BlockSpec((_D, _D), lambda i: (0, 0)),
            pl.BlockSpec((_NC, _R, 1), lambda i: (0, i, 0)),
        ],
        out_specs=[
            pl.BlockSpec((_R, _D), lambda i: (i, 0)),
            pl.BlockSpec((_R, 1), lambda i: (i, 0)),
        ],
        out_shape=[
            jax.ShapeDtypeStruct((_NN, _D), jnp.float32),
            jax.ShapeDtypeStruct((_NN, 1), jnp.float32),
        ],
    )(x, W1, degp)


def _tc2_body(s_ref, g_ref, dinv_ref, w_ref, g2_ref):
    dinv = dinv_ref[...]
    p = (s_ref[0] + s_ref[1] + g_ref[...]) * dinv
    g2_ref[...] = _dot(p, w_ref[...]) * dinv


def _tc_layer2(s, g, dinv, W2):
    return pl.pallas_call(
        _tc2_body,
        grid=(_NN // _R,),
        in_specs=[
            pl.BlockSpec((_NC, _R, _D), lambda i: (0, i, 0)),
            pl.BlockSpec((_R, _D), lambda i: (i, 0)),
            pl.BlockSpec((_R, 1), lambda i: (i, 0)),
            pl.BlockSpec((_D, _D), lambda i: (0, 0)),
        ],
        out_specs=pl.BlockSpec((_R, _D), lambda i: (i, 0)),
        out_shape=jax.ShapeDtypeStruct((_NN, _D), jnp.float32),
    )(s, g, dinv, W2)


def _tc3_body(s_ref, g_ref, dinv_ref, o_ref):
    o_ref[...] = (s_ref[0] + s_ref[1] + g_ref[...]) * dinv_ref[...]


def _tc_layer3(s, g, dinv):
    return pl.pallas_call(
        _tc3_body,
        grid=(_NN // _R,),
        in_specs=[
            pl.BlockSpec((_NC, _R, _D), lambda i: (0, i, 0)),
            pl.BlockSpec((_R, _D), lambda i: (i, 0)),
            pl.BlockSpec((_R, 1), lambda i: (i, 0)),
        ],
        out_specs=pl.BlockSpec((_R, _D), lambda i: (i, 0)),
        out_shape=jax.ShapeDtypeStruct((_NN, _D), jnp.float32),
    )(s, g, dinv)


def kernel(x, edge_index, W1, W2):
    src = edge_index[0]
    dst = edge_index[1]
    pidx = jnp.arange(_EP - _NE, dtype=jnp.int32)
    src_p = jnp.concatenate([src, pidx % _NN]).reshape(_EP // _K, _K)
    dst_p = jnp.concatenate([dst, _NN + pidx % (_NP - _NN)]).reshape(
        _EP // _K, _K)

    degp = _sc_degree(dst_p).reshape(_NC, _NP, 1)
    g1, dinv = _tc_scale(x, W1, degp)

    s1 = _sc_scatter(g1, src_p, dst_p)
    g2 = _tc_layer2(s1, g1, dinv, W2)

    s2 = _sc_scatter(g2, src_p, dst_p)
    return _tc_layer3(s2, g2, dinv)
